# baseline (speedup 1.0000x reference)
import jax
import jax.numpy as jnp
from jax import lax
from jax.experimental import pallas as pl
from jax.experimental.pallas import tpu as pltpu
from jax.experimental.pallas import tpu_sc as plsc

N = 10000
E = 320000
D = 128
DH = 64
TOL = 3e-06
MAX_ITER = 50
PHANTOM_GRAD = 5

NC = 2
NS = 16
CHUNK = 128
NP = 10240
NODES_PER_TILE = NP // NS
QGRP = 40
NCHUNK_CAP = (E + NS * (CHUNK - 1)) // CHUNK + NS + QGRP
E_SLOT = NCHUNK_CAP * CHUNK
UCHUNK = 64
NUCH = NODES_PER_TILE // UCHUNK

_f32 = jnp.float32
_i32 = jnp.int32


def _sc_step_body(z_hbm, xb_hbm, srcq_hbm, dstq_hbm, wq_hbm, bg_hbm,
                  meta_hbm,
                  znew_hbm, err_hbm,
                  srcbb, dstbb, wbb, zsrc, msg, accv, zv, xbv, errb, bgv,
                  metav, acc_sh, ssem):
    c = lax.axis_index("c")
    s = lax.axis_index("s")

    pltpu.sync_copy(bg_hbm, bgv)
    bsv = bgv[pl.ds(0, 16)]
    gsv = bgv[pl.ds(16, 16)]

    pltpu.sync_copy(meta_hbm, metav)
    lane = lax.iota(_i32, 16)
    m = lane == s
    offt = jnp.sum(jnp.where(m, metav[pl.ds(0, 16)], 0))
    nct = jnp.sum(jnp.where(m, metav[pl.ds(16, 16)], 0))

    def _zero_row(r, _):
        for g in range(DH // 16):
            zv[r, pl.ds(g * 16, 16)] = jnp.zeros((16,), _f32)
        return 0
    lax.fori_loop(0, UCHUNK, _zero_row, 0)
    for k in range(NUCH):
        pltpu.sync_copy(zv, acc_sh.at[pl.ds(s * NODES_PER_TILE + k * UCHUNK,
                                            UCHUNK)])
    pltpu.sync_copy(z_hbm.at[pl.ds(c * NP + s * NODES_PER_TILE,
                                   NODES_PER_TILE)], zsrc)
    plsc.subcore_barrier()

    def _fire_scatter(g, b):
        pltpu.async_copy(msg.at[b], acc_sh.at[dstbb.at[g]], ssem.at[b],
                         add=True)

    def _drain_scatter(b):
        pltpu.make_async_copy(msg.at[b], acc_sh.at[dstbb.at[0]],
                              ssem.at[b]).wait()

    _dnums = lax.GatherDimensionNumbers(
        offset_dims=(), collapsed_slice_dims=(0,), start_index_map=(0,))

    def _bcast(v16, t):
        return lax.gather(v16, jnp.full((16, 1), t, _i32), _dnums, (1,),
                          mode=lax.GatherScatterMode.PROMISE_IN_BOUNDS)

    def _mult(g, b):
        @plsc.parallel_loop(0, CHUNK // 16, 1, unroll=2)
        def _q(q):
            wv16 = wbb[g, pl.ds(q * 16, 16)]
            sv16 = srcbb[g, pl.ds(q * 16, 16)]
            for t in range(16):
                wsp = _bcast(wv16, t)
                ssp = _bcast(sv16, t)
                e = q * 16 + t
                for f in range(DH // 16):
                    colv = lax.iota(_i32, 16) + (f * 16)
                    vals = plsc.load_gather(zsrc, [ssp, colv])
                    msg[b, e, pl.ds(f * 16, 16)] = vals * wsp

    def _proc(g, b):
        @pl.when(g >= 2)
        def _():
            _drain_scatter(b)
        _mult(g, b)
        _fire_scatter(g, b)

    nst = (nct + QGRP - 1) // QGRP

    def _stage(stg, _):
        base = offt + stg * QGRP
        pltpu.sync_copy(srcq_hbm.at[pl.ds(base, QGRP)], srcbb)
        pltpu.sync_copy(dstq_hbm.at[pl.ds(base, QGRP)], dstbb)
        pltpu.sync_copy(wq_hbm.at[pl.ds(base, QGRP)], wbb)
        nloc = jnp.minimum(QGRP, nct - stg * QGRP)

        def _chunk(g, _):
            @pl.when(g % 2 == 0)
            def _():
                _proc(g, 0)

            @pl.when(g % 2 == 1)
            def _():
                _proc(g, 1)
            return 0
        lax.fori_loop(0, nloc, _chunk, 0)

        @pl.when(nloc >= 1)
        def _():
            _drain_scatter(0)

        @pl.when(nloc >= 2)
        def _():
            _drain_scatter(1)
        return 0
    lax.fori_loop(0, nst, _stage, 0)
    plsc.subcore_barrier()

    def _upd(k, errv):
        row0 = s * NODES_PER_TILE + k * UCHUNK
        pltpu.sync_copy(acc_sh.at[pl.ds(row0, UCHUNK)], accv)
        pltpu.sync_copy(z_hbm.at[pl.ds(c * NP + row0, UCHUNK)], zv)
        pltpu.sync_copy(xb_hbm.at[pl.ds(c * NP + row0, UCHUNK)], xbv)

        def _row(r, ev):
            for g in range(DH // 16):
                sl = pl.ds(g * 16, 16)
                a = accv[r, sl]
                zz = zv[r, sl]
                xx = xbv[r, sl]
                zn = (1.0 - bsv) * zz + bsv * jnp.maximum(gsv * a + xx, 0.0)
                zv[r, sl] = zn
                dd = zn - zz
                ev = ev + dd * dd
            return ev
        errv = lax.fori_loop(0, UCHUNK, _row, errv)
        pltpu.sync_copy(zv, znew_hbm.at[pl.ds(c * NP + row0, UCHUNK)])
        return errv

    errv = lax.fori_loop(0, NUCH, _upd, jnp.zeros((16,), _f32))
    errb[...] = errv
    pltpu.sync_copy(errb, err_hbm.at[pl.ds((c * NS + s) * 16, 16)])


@jax.jit
def _sc_step(z, xb, srcq, dstq, wq, bg, meta):
    mesh = plsc.VectorSubcoreMesh(core_axis_name="c", subcore_axis_name="s")
    return pl.kernel(
        _sc_step_body,
        out_type=(
            jax.ShapeDtypeStruct((2 * NP, DH), _f32),
            jax.ShapeDtypeStruct((NC * NS * 16,), _f32),
        ),
        mesh=mesh,
        compiler_params=pltpu.CompilerParams(use_tc_tiling_on_sc=False,
                                             needs_layout_passes=False),
        scratch_types=[
            pltpu.VMEM((QGRP, CHUNK), _i32),
            pltpu.VMEM((QGRP, CHUNK), _i32),
            pltpu.VMEM((QGRP, CHUNK), _f32),
            pltpu.VMEM((NODES_PER_TILE, DH), _f32),
            pltpu.VMEM((2, CHUNK, DH), _f32),
            pltpu.VMEM((UCHUNK, DH), _f32),
            pltpu.VMEM((UCHUNK, DH), _f32),
            pltpu.VMEM((UCHUNK, DH), _f32),
            pltpu.VMEM((16,), _f32),
            pltpu.VMEM((32,), _f32),
            pltpu.VMEM((32,), _i32),
            pltpu.VMEM_SHARED((NP, DH), _f32),
            pltpu.SemaphoreType.DMA((2,)),
        ],
    )(z, xb, srcq, dstq, wq, bg, meta)


def _enc_body(x_ref, wenc_ref, wbias_ref, b_ref, out_ref):
    h = jnp.dot(x_ref[...], wenc_ref[...], preferred_element_type=_f32)
    out_ref[...] = (
        jnp.dot(h, wbias_ref[...], preferred_element_type=_f32) + b_ref[...]
    )


@jax.jit
def _encoder(x, wenc_t, wbias_t, b):
    blk = 1000
    grid = N // blk
    return pl.pallas_call(
        _enc_body,
        grid=(grid,),
        in_specs=[
            pl.BlockSpec((blk, D), lambda i: (i, 0)),
            pl.BlockSpec((D, D), lambda i: (0, 0)),
            pl.BlockSpec((D, D), lambda i: (0, 0)),
            pl.BlockSpec((1, D), lambda i: (0, 0)),
        ],
        out_specs=pl.BlockSpec((blk, D), lambda i: (i, 0)),
        out_shape=jax.ShapeDtypeStruct((N, D), _f32),
    )(x, wenc_t, wbias_t, b)


def _dec_body(z0_ref, z1_ref, wdec_ref, b_ref, out_ref):
    h0 = jnp.maximum(z0_ref[...], 0.0)
    h1 = jnp.maximum(z1_ref[...], 0.0)
    out_ref[...] = (
        jnp.dot(h0, wdec_ref[:DH, :], preferred_element_type=_f32)
        + jnp.dot(h1, wdec_ref[DH:, :], preferred_element_type=_f32)
        + b_ref[...]
    )


@jax.jit
def _decoder(z0h, z1h, wdec_t, b):
    blk = 1000
    grid = N // blk
    return pl.pallas_call(
        _dec_body,
        grid=(grid,),
        in_specs=[
            pl.BlockSpec((blk, DH), lambda i: (i, 0)),
            pl.BlockSpec((blk, DH), lambda i: (i, 0)),
            pl.BlockSpec((D, D), lambda i: (0, 0)),
            pl.BlockSpec((1, D), lambda i: (0, 0)),
        ],
        out_specs=pl.BlockSpec((blk, D), lambda i: (i, 0)),
        out_shape=jax.ShapeDtypeStruct((N, D), _f32),
    )(z0h, z1h, wdec_t, b)


def kernel(x, edge_index, edge_weight, W_enc, W_bias, b_bias, W_dec, b_dec,
           beta, gamma):
    beta_s = jax.nn.sigmoid(beta)
    gamma_s = jax.nn.sigmoid(gamma)

    xb = _encoder(x, W_enc.T, W_bias.T, b_bias.reshape(1, D))
    xbflat = jnp.zeros((2 * NP, DH), _f32)
    xbflat = xbflat.at[0:N].set(xb[:, :DH]).at[NP:NP + N].set(xb[:, DH:])

    src = edge_index[0]
    dst = edge_index[1]
    w = edge_weight

    tile = src // NODES_PER_TILE
    order = jnp.argsort(tile)
    tile_s = tile[order]
    src_s = src[order] - tile_s * NODES_PER_TILE
    dst_s = dst[order]
    w_s = w[order]
    counts = jnp.bincount(tile, length=NS).astype(_i32)
    starts = jnp.concatenate([jnp.zeros((1,), _i32),
                              jnp.cumsum(counts)[:-1].astype(_i32)])
    ncc = (counts + CHUNK - 1) // CHUNK
    choff = jnp.concatenate([jnp.zeros((1,), _i32),
                             jnp.cumsum(ncc)[:-1].astype(_i32)])
    rank = jnp.arange(E, dtype=_i32) - starts[tile_s]
    slot = choff[tile_s] * CHUNK + rank
    srcq = jnp.zeros((E_SLOT,), _i32).at[slot].set(src_s).reshape(-1, CHUNK)
    dstq = jnp.zeros((E_SLOT,), _i32).at[slot].set(dst_s).reshape(-1, CHUNK)
    wq = jnp.zeros((E_SLOT,), _f32).at[slot].set(w_s).reshape(-1, CHUNK)
    meta = jnp.concatenate([choff, ncc])

    bg = jnp.concatenate([jnp.broadcast_to(beta_s, (16,)),
                          jnp.broadcast_to(gamma_s, (16,))]).astype(_f32)

    def step(z):
        znew, errparts = _sc_step(z, xbflat, srcq, dstq, wq, bg, meta)
        return znew, jnp.sum(errparts)

    z0 = jnp.zeros((2 * NP, DH), _f32)
    z1, e1 = step(z0)

    def cond(state):
        _, errsq, it = state
        return jnp.logical_and(errsq > jnp.float32(TOL) * jnp.float32(TOL),
                               it < MAX_ITER)

    def body(state):
        z, _, it = state
        znew, errsq = step(z)
        return (znew, errsq, it + 1)

    z_star, _, _ = lax.while_loop(cond, body, (z1, e1, jnp.int32(1)))

    z = z_star
    for _ in range(PHANTOM_GRAD):
        z, _ = step(z)

    return _decoder(z[0:N], z[NP:NP + N], W_dec.T, b_dec.reshape(1, D))


# speedup vs baseline: 1.5420x; 1.5420x over previous
import jax
import jax.numpy as jnp
from jax import lax
from jax.experimental import pallas as pl
from jax.experimental.pallas import tpu as pltpu
from jax.experimental.pallas import tpu_sc as plsc

N = 10000
E = 320000
D = 128
DH = 64
TOL = 3e-06
MAX_ITER = 50
PHANTOM_GRAD = 5

NC = 2
NS = 16
CHUNK = 128
NP = 10240
NODES_PER_TILE = NP // NS
QGRP = 40
NCHUNK_CAP = (E + NS * (CHUNK - 1)) // CHUNK + NS + QGRP
E_SLOT = NCHUNK_CAP * CHUNK
UCHUNK = 64
NUCH = NODES_PER_TILE // UCHUNK

_f32 = jnp.float32
_i32 = jnp.int32


def _sc_step_body(z_hbm, xb_hbm, srcq_hbm, dstq_hbm, wq_hbm, bg_hbm,
                  meta_hbm,
                  znew_hbm, err_hbm,
                  srcbb, dstbb, wbb, zsrc, msg, accv, zv, xbv, errb, bgv,
                  metav, acc_sh, ssem):
    c = lax.axis_index("c")
    s = lax.axis_index("s")

    pltpu.sync_copy(bg_hbm, bgv)
    bsv = bgv[pl.ds(0, 16)]
    gsv = bgv[pl.ds(16, 16)]

    pltpu.sync_copy(meta_hbm.at[s], metav)
    mrow = metav[pl.ds(0, 16)]
    offt = mrow[0]
    nct = mrow[1]

    def _zero_row(r, _):
        for g in range(DH // 16):
            zv[r, pl.ds(g * 16, 16)] = jnp.zeros((16,), _f32)
        return 0
    lax.fori_loop(0, UCHUNK, _zero_row, 0)
    for k in range(NUCH):
        pltpu.sync_copy(zv, acc_sh.at[pl.ds(s * NODES_PER_TILE + k * UCHUNK,
                                            UCHUNK)])
    pltpu.sync_copy(z_hbm.at[pl.ds(c * NP + s * NODES_PER_TILE,
                                   NODES_PER_TILE)], zsrc)
    plsc.subcore_barrier()

    def _fire_scatter(g, b):
        pltpu.async_copy(msg.at[b], acc_sh.at[dstbb.at[g]], ssem.at[b],
                         add=True)

    def _drain_scatter(b):
        pltpu.make_async_copy(msg.at[b], acc_sh.at[dstbb.at[0]],
                              ssem.at[b]).wait()

    def _mult(g, b):
        @plsc.parallel_loop(0, CHUNK // 16, 1, unroll=2)
        def _q(q):
            sv16 = srcbb[g, pl.ds(q * 16, 16)]
            wv16 = wbb[g, pl.ds(q * 16, 16)]
            for t in range(16):
                srcl = sv16[t]
                wsp = jnp.broadcast_to(wv16[t], (16,))
                e = q * 16 + t
                for f in range(DH // 16):
                    sl = pl.ds(f * 16, 16)
                    msg[b, e, sl] = zsrc[srcl, sl] * wsp

    def _proc(g, b):
        @pl.when(g >= 2)
        def _():
            _drain_scatter(b)
        _mult(g, b)
        _fire_scatter(g, b)

    nst = (nct + QGRP - 1) // QGRP

    def _stage(stg, _):
        base = offt + stg * QGRP
        pltpu.sync_copy(srcq_hbm.at[pl.ds(base, QGRP)], srcbb)
        pltpu.sync_copy(dstq_hbm.at[pl.ds(base, QGRP)], dstbb)
        pltpu.sync_copy(wq_hbm.at[pl.ds(base, QGRP)], wbb)
        nloc = jnp.minimum(QGRP, nct - stg * QGRP)

        def _chunk(g, _):
            @pl.when(g % 2 == 0)
            def _():
                _proc(g, 0)

            @pl.when(g % 2 == 1)
            def _():
                _proc(g, 1)
            return 0
        lax.fori_loop(0, nloc, _chunk, 0)

        @pl.when(nloc >= 1)
        def _():
            _drain_scatter(0)

        @pl.when(nloc >= 2)
        def _():
            _drain_scatter(1)
        return 0
    lax.fori_loop(0, nst, _stage, 0)
    plsc.subcore_barrier()

    def _upd(k, errv):
        row0 = s * NODES_PER_TILE + k * UCHUNK
        pltpu.sync_copy(acc_sh.at[pl.ds(row0, UCHUNK)], accv)
        pltpu.sync_copy(z_hbm.at[pl.ds(c * NP + row0, UCHUNK)], zv)
        pltpu.sync_copy(xb_hbm.at[pl.ds(c * NP + row0, UCHUNK)], xbv)

        def _row(r, ev):
            for g in range(DH // 16):
                sl = pl.ds(g * 16, 16)
                a = accv[r, sl]
                zz = zv[r, sl]
                xx = xbv[r, sl]
                zn = (1.0 - bsv) * zz + bsv * jnp.maximum(gsv * a + xx, 0.0)
                zv[r, sl] = zn
                dd = zn - zz
                ev = ev + dd * dd
            return ev
        errv = lax.fori_loop(0, UCHUNK, _row, errv)
        pltpu.sync_copy(zv, znew_hbm.at[pl.ds(c * NP + row0, UCHUNK)])
        return errv

    errv = lax.fori_loop(0, NUCH, _upd, jnp.zeros((16,), _f32))
    errb[...] = errv
    pltpu.sync_copy(errb, err_hbm.at[pl.ds((c * NS + s) * 16, 16)])


@jax.jit
def _sc_step(z, xb, srcq, dstq, wq, bg, meta):
    mesh = plsc.VectorSubcoreMesh(core_axis_name="c", subcore_axis_name="s")
    return pl.kernel(
        _sc_step_body,
        out_type=(
            jax.ShapeDtypeStruct((2 * NP, DH), _f32),
            jax.ShapeDtypeStruct((NC * NS * 16,), _f32),
        ),
        mesh=mesh,
        compiler_params=pltpu.CompilerParams(use_tc_tiling_on_sc=False),
        scratch_types=[
            pltpu.VMEM((QGRP, CHUNK), _i32),
            pltpu.VMEM((QGRP, CHUNK), _i32),
            pltpu.VMEM((QGRP, CHUNK), _f32),
            pltpu.VMEM((NODES_PER_TILE, DH), _f32),
            pltpu.VMEM((2, CHUNK, DH), _f32),
            pltpu.VMEM((UCHUNK, DH), _f32),
            pltpu.VMEM((UCHUNK, DH), _f32),
            pltpu.VMEM((UCHUNK, DH), _f32),
            pltpu.VMEM((16,), _f32),
            pltpu.VMEM((32,), _f32),
            pltpu.VMEM((16,), _i32),
            pltpu.VMEM_SHARED((NP, DH), _f32),
            pltpu.SemaphoreType.DMA((2,)),
        ],
    )(z, xb, srcq, dstq, wq, bg, meta)


def _enc_body(x_ref, wenc_ref, wbias_ref, b_ref, out_ref):
    h = jnp.dot(x_ref[...], wenc_ref[...], preferred_element_type=_f32)
    out_ref[...] = (
        jnp.dot(h, wbias_ref[...], preferred_element_type=_f32) + b_ref[...]
    )


@jax.jit
def _encoder(x, wenc_t, wbias_t, b):
    blk = 1000
    grid = N // blk
    return pl.pallas_call(
        _enc_body,
        grid=(grid,),
        in_specs=[
            pl.BlockSpec((blk, D), lambda i: (i, 0)),
            pl.BlockSpec((D, D), lambda i: (0, 0)),
            pl.BlockSpec((D, D), lambda i: (0, 0)),
            pl.BlockSpec((1, D), lambda i: (0, 0)),
        ],
        out_specs=pl.BlockSpec((blk, D), lambda i: (i, 0)),
        out_shape=jax.ShapeDtypeStruct((N, D), _f32),
    )(x, wenc_t, wbias_t, b)


def _dec_body(z0_ref, z1_ref, wdec_ref, b_ref, out_ref):
    h0 = jnp.maximum(z0_ref[...], 0.0)
    h1 = jnp.maximum(z1_ref[...], 0.0)
    out_ref[...] = (
        jnp.dot(h0, wdec_ref[:DH, :], preferred_element_type=_f32)
        + jnp.dot(h1, wdec_ref[DH:, :], preferred_element_type=_f32)
        + b_ref[...]
    )


@jax.jit
def _decoder(z0h, z1h, wdec_t, b):
    blk = 1000
    grid = N // blk
    return pl.pallas_call(
        _dec_body,
        grid=(grid,),
        in_specs=[
            pl.BlockSpec((blk, DH), lambda i: (i, 0)),
            pl.BlockSpec((blk, DH), lambda i: (i, 0)),
            pl.BlockSpec((D, D), lambda i: (0, 0)),
            pl.BlockSpec((1, D), lambda i: (0, 0)),
        ],
        out_specs=pl.BlockSpec((blk, D), lambda i: (i, 0)),
        out_shape=jax.ShapeDtypeStruct((N, D), _f32),
    )(z0h, z1h, wdec_t, b)


def kernel(x, edge_index, edge_weight, W_enc, W_bias, b_bias, W_dec, b_dec,
           beta, gamma):
    beta_s = jax.nn.sigmoid(beta)
    gamma_s = jax.nn.sigmoid(gamma)

    xb = _encoder(x, W_enc.T, W_bias.T, b_bias.reshape(1, D))
    xbflat = jnp.zeros((2 * NP, DH), _f32)
    xbflat = xbflat.at[0:N].set(xb[:, :DH]).at[NP:NP + N].set(xb[:, DH:])

    src = edge_index[0]
    dst = edge_index[1]
    w = edge_weight

    tile = src // NODES_PER_TILE
    order = jnp.argsort(tile)
    tile_s = tile[order]
    src_s = src[order] - tile_s * NODES_PER_TILE
    dst_s = dst[order]
    w_s = w[order]
    counts = jnp.bincount(tile, length=NS).astype(_i32)
    starts = jnp.concatenate([jnp.zeros((1,), _i32),
                              jnp.cumsum(counts)[:-1].astype(_i32)])
    ncc = (counts + CHUNK - 1) // CHUNK
    choff = jnp.concatenate([jnp.zeros((1,), _i32),
                             jnp.cumsum(ncc)[:-1].astype(_i32)])
    rank = jnp.arange(E, dtype=_i32) - starts[tile_s]
    slot = choff[tile_s] * CHUNK + rank
    srcq = jnp.zeros((E_SLOT,), _i32).at[slot].set(src_s).reshape(-1, CHUNK)
    dstq = jnp.zeros((E_SLOT,), _i32).at[slot].set(dst_s).reshape(-1, CHUNK)
    wq = jnp.zeros((E_SLOT,), _f32).at[slot].set(w_s).reshape(-1, CHUNK)
    meta = jnp.zeros((NS, 16), _i32).at[:, 0].set(choff).at[:, 1].set(ncc)

    bg = jnp.concatenate([jnp.broadcast_to(beta_s, (16,)),
                          jnp.broadcast_to(gamma_s, (16,))]).astype(_f32)

    def step(z):
        znew, errparts = _sc_step(z, xbflat, srcq, dstq, wq, bg, meta)
        return znew, jnp.sum(errparts)

    z0 = jnp.zeros((2 * NP, DH), _f32)
    z1, e1 = step(z0)

    def cond(state):
        _, errsq, it = state
        return jnp.logical_and(errsq > jnp.float32(TOL) * jnp.float32(TOL),
                               it < MAX_ITER)

    def body(state):
        z, _, it = state
        znew, errsq = step(z)
        return (znew, errsq, it + 1)

    z_star, _, _ = lax.while_loop(cond, body, (z1, e1, jnp.int32(1)))

    z = z_star
    for _ in range(PHANTOM_GRAD):
        z, _ = step(z)

    return _decoder(z[0:N], z[NP:NP + N], W_dec.T, b_dec.reshape(1, D))


# speedup vs baseline: 2.2678x; 1.4707x over previous
import jax
import jax.numpy as jnp
from jax import lax
from jax.experimental import pallas as pl
from jax.experimental.pallas import tpu as pltpu
from jax.experimental.pallas import tpu_sc as plsc

N = 10000
E = 320000
D = 128
DH = 64
TOL = 3e-06
MAX_ITER = 50
PHANTOM_GRAD = 5

NC = 2
NS = 16
CHUNK = 128
EPT = 20480
CPT = EPT // CHUNK
NSTAGE = 8
QGRP = CPT // NSTAGE
NBUF = 4
AHEAD = 2
E_PAD = NS * EPT
NP = 10240
NODES_PER_TILE = NP // NS
UCHUNK = 32
NUCH = NODES_PER_TILE // UCHUNK

_f32 = jnp.float32


def _sc_step_body(z_hbm, xb_hbm, srcq_hbm, dstq_hbm, wq_hbm, bg_hbm,
                  znew_hbm, err_hbm,
                  srcbb, dstbb, wbb, rows, accv, zv, xbv, errb, bgv, acc_sh,
                  zsp, gsem, ssem):
    c = lax.axis_index("c")
    s = lax.axis_index("s")

    pltpu.sync_copy(bg_hbm, bgv)
    bsv = bgv[pl.ds(0, 16)]
    gsv = bgv[pl.ds(16, 16)]

    def _zero_row(r, _):
        for g in range(DH // 16):
            zv[r, pl.ds(g * 16, 16)] = jnp.zeros((16,), _f32)
        return 0
    lax.fori_loop(0, UCHUNK, _zero_row, 0)
    for k in range(NUCH):
        pltpu.sync_copy(zv, acc_sh.at[pl.ds(s * NODES_PER_TILE + k * UCHUNK,
                                            UCHUNK)])
    pltpu.sync_copy(z_hbm.at[pl.ds(c * NP + s * NODES_PER_TILE,
                                   NODES_PER_TILE)],
                    zsp.at[pl.ds(s * NODES_PER_TILE, NODES_PER_TILE)])
    plsc.subcore_barrier()

    def _fire_gather(g, b):
        pltpu.async_copy(zsp.at[srcbb.at[g]], rows.at[b], gsem.at[b])

    def _drain_gather(g, b):
        pltpu.make_async_copy(zsp.at[srcbb.at[g]], rows.at[b],
                              gsem.at[b]).wait()

    def _fire_scatter(g, b):
        pltpu.async_copy(rows.at[b], acc_sh.at[dstbb.at[g]], ssem.at[b],
                         add=True)

    def _drain_scatter(b):
        pltpu.make_async_copy(rows.at[b], acc_sh.at[dstbb.at[0]],
                              ssem.at[b]).wait()

    _dnums = lax.GatherDimensionNumbers(
        offset_dims=(), collapsed_slice_dims=(0,), start_index_map=(0,))

    def _mult(g, b):
        @plsc.parallel_loop(0, CHUNK // 16, 1, unroll=2)
        def _q(q):
            wv16 = wbb[g, pl.ds(q * 16, 16)]
            for t in range(16):
                wsp = lax.gather(
                    wv16, jnp.full((16, 1), t, jnp.int32), _dnums, (1,),
                    mode=lax.GatherScatterMode.PROMISE_IN_BOUNDS)
                e = q * 16 + t
                for f in range(DH // 16):
                    sl = pl.ds(f * 16, 16)
                    rows[b, e, sl] = rows[b, e, sl] * wsp

    for h in range(NSTAGE):
        base = s * CPT + h * QGRP
        pltpu.sync_copy(srcq_hbm.at[pl.ds(base, QGRP)], srcbb)
        pltpu.sync_copy(dstq_hbm.at[pl.ds(base, QGRP)], dstbb)
        pltpu.sync_copy(wq_hbm.at[pl.ds(base, QGRP)], wbb)
        for g0 in range(AHEAD):
            _fire_gather(g0, g0 % NBUF)

        @pl.loop(0, QGRP, step=NBUF)
        def _outer(gg):
            for b in range(NBUF):
                g = gg + b
                ga = g + AHEAD
                b2 = (b + AHEAD) % NBUF

                @pl.when(ga < QGRP)
                def _():
                    @pl.when(ga - NBUF >= 0)
                    def _():
                        _drain_scatter(b2)
                    _fire_gather(ga, b2)

                _drain_gather(g, b)
                _mult(g, b)
                _fire_scatter(g, b)

        for b in range(NBUF):
            _drain_scatter(b)
    plsc.subcore_barrier()

    def _upd(k, errv):
        row0 = s * NODES_PER_TILE + k * UCHUNK
        pltpu.sync_copy(acc_sh.at[pl.ds(row0, UCHUNK)], accv)
        pltpu.sync_copy(z_hbm.at[pl.ds(c * NP + row0, UCHUNK)], zv)
        pltpu.sync_copy(xb_hbm.at[pl.ds(c * NP + row0, UCHUNK)], xbv)

        def _row(r, ev):
            for g in range(DH // 16):
                sl = pl.ds(g * 16, 16)
                a = accv[r, sl]
                zz = zv[r, sl]
                xx = xbv[r, sl]
                zn = (1.0 - bsv) * zz + bsv * jnp.maximum(gsv * a + xx, 0.0)
                zv[r, sl] = zn
                dd = zn - zz
                ev = ev + dd * dd
            return ev
        errv = lax.fori_loop(0, UCHUNK, _row, errv)
        pltpu.sync_copy(zv, znew_hbm.at[pl.ds(c * NP + row0, UCHUNK)])
        return errv

    errv = lax.fori_loop(0, NUCH, _upd, jnp.zeros((16,), _f32))
    errb[...] = errv
    pltpu.sync_copy(errb, err_hbm.at[pl.ds((c * NS + s) * 16, 16)])


@jax.jit
def _sc_step(z, xb, srcq, dstq, wq, bg):
    mesh = plsc.VectorSubcoreMesh(core_axis_name="c", subcore_axis_name="s")
    return pl.kernel(
        _sc_step_body,
        out_type=(
            jax.ShapeDtypeStruct((2 * NP, DH), _f32),
            jax.ShapeDtypeStruct((NC * NS * 16,), _f32),
        ),
        mesh=mesh,
        compiler_params=pltpu.CompilerParams(use_tc_tiling_on_sc=False),
        scratch_types=[
            pltpu.VMEM((QGRP, CHUNK), jnp.int32),
            pltpu.VMEM((QGRP, CHUNK), jnp.int32),
            pltpu.VMEM((QGRP, CHUNK), _f32),
            pltpu.VMEM((NBUF, CHUNK, DH), _f32),
            pltpu.VMEM((UCHUNK, DH), _f32),
            pltpu.VMEM((UCHUNK, DH), _f32),
            pltpu.VMEM((UCHUNK, DH), _f32),
            pltpu.VMEM((16,), _f32),
            pltpu.VMEM((32,), _f32),
            pltpu.VMEM_SHARED((NP, DH), _f32),
            pltpu.VMEM_SHARED((NP, DH), _f32),
            pltpu.SemaphoreType.DMA((NBUF,)),
            pltpu.SemaphoreType.DMA((NBUF,)),
        ],
    )(z, xb, srcq, dstq, wq, bg)


def _enc_body(x_ref, wenc_ref, wbias_ref, b_ref, out_ref):
    h = jnp.dot(x_ref[...], wenc_ref[...], preferred_element_type=_f32)
    out_ref[...] = (
        jnp.dot(h, wbias_ref[...], preferred_element_type=_f32) + b_ref[...]
    )


@jax.jit
def _encoder(x, wenc_t, wbias_t, b):
    blk = 1000
    grid = N // blk
    return pl.pallas_call(
        _enc_body,
        grid=(grid,),
        in_specs=[
            pl.BlockSpec((blk, D), lambda i: (i, 0)),
            pl.BlockSpec((D, D), lambda i: (0, 0)),
            pl.BlockSpec((D, D), lambda i: (0, 0)),
            pl.BlockSpec((1, D), lambda i: (0, 0)),
        ],
        out_specs=pl.BlockSpec((blk, D), lambda i: (i, 0)),
        out_shape=jax.ShapeDtypeStruct((N, D), _f32),
    )(x, wenc_t, wbias_t, b)


def _dec_body(z0_ref, z1_ref, wdec_ref, b_ref, out_ref):
    h0 = jnp.maximum(z0_ref[...], 0.0)
    h1 = jnp.maximum(z1_ref[...], 0.0)
    out_ref[...] = (
        jnp.dot(h0, wdec_ref[:DH, :], preferred_element_type=_f32)
        + jnp.dot(h1, wdec_ref[DH:, :], preferred_element_type=_f32)
        + b_ref[...]
    )


@jax.jit
def _decoder(z0h, z1h, wdec_t, b):
    blk = 1000
    grid = N // blk
    return pl.pallas_call(
        _dec_body,
        grid=(grid,),
        in_specs=[
            pl.BlockSpec((blk, DH), lambda i: (i, 0)),
            pl.BlockSpec((blk, DH), lambda i: (i, 0)),
            pl.BlockSpec((D, D), lambda i: (0, 0)),
            pl.BlockSpec((1, D), lambda i: (0, 0)),
        ],
        out_specs=pl.BlockSpec((blk, D), lambda i: (i, 0)),
        out_shape=jax.ShapeDtypeStruct((N, D), _f32),
    )(z0h, z1h, wdec_t, b)


def kernel(x, edge_index, edge_weight, W_enc, W_bias, b_bias, W_dec, b_dec,
           beta, gamma):
    beta_s = jax.nn.sigmoid(beta)
    gamma_s = jax.nn.sigmoid(gamma)

    xb = _encoder(x, W_enc.T, W_bias.T, b_bias.reshape(1, D))
    xbflat = jnp.zeros((2 * NP, DH), _f32)
    xbflat = xbflat.at[0:N].set(xb[:, :DH]).at[NP:NP + N].set(xb[:, DH:])

    src = edge_index[0]
    dst = edge_index[1]
    pad = E_PAD - E
    srcp = jnp.concatenate([src, jnp.zeros((pad,), jnp.int32)])
    dstp = jnp.concatenate([dst, jnp.zeros((pad,), jnp.int32)])
    wp = jnp.concatenate([edge_weight, jnp.zeros((pad,), _f32)])
    srcq = srcp.reshape(NS * CPT, CHUNK)
    dstq = dstp.reshape(NS * CPT, CHUNK)
    wq = wp.reshape(NS * CPT, CHUNK)
    bg = jnp.concatenate([jnp.broadcast_to(beta_s, (16,)),
                          jnp.broadcast_to(gamma_s, (16,))]).astype(_f32)

    def step(z):
        znew, errparts = _sc_step(z, xbflat, srcq, dstq, wq, bg)
        return znew, jnp.sum(errparts)

    z0 = jnp.zeros((2 * NP, DH), _f32)
    z1, e1 = step(z0)

    def cond(state):
        _, errsq, it = state
        return jnp.logical_and(errsq > jnp.float32(TOL) * jnp.float32(TOL),
                               it < MAX_ITER)

    def body(state):
        z, _, it = state
        znew, errsq = step(z)
        return (znew, errsq, it + 1)

    z_star, _, _ = lax.while_loop(cond, body, (z1, e1, jnp.int32(1)))

    z = z_star
    for _ in range(PHANTOM_GRAD):
        z, _ = step(z)

    return _decoder(z[0:N], z[NP:NP + N], W_dec.T, b_dec.reshape(1, D))


# speedup vs baseline: 2.4188x; 1.0666x over previous
import jax
import jax.numpy as jnp
from jax import lax
from jax.experimental import pallas as pl
from jax.experimental.pallas import tpu as pltpu
from jax.experimental.pallas import tpu_sc as plsc

N = 10000
E = 320000
D = 128
DH = 64
TOL = 3e-06
MAX_ITER = 50
PHANTOM_GRAD = 5

NC = 2
NS = 16
CHUNK = 128
EPT = 20480
CPT = EPT // CHUNK
NSTAGE = 8
QGRP = CPT // NSTAGE
NBUF = 4
AHEAD = 2
E_PAD = NS * EPT
NP = 10240
NODES_PER_TILE = NP // NS
UCHUNK = 32
NUCH = NODES_PER_TILE // UCHUNK

_f32 = jnp.float32


def _sc_step_body(z_hbm, xb_hbm, srcq_hbm, dstq_hbm, wq_hbm, bg_hbm,
                  znew_hbm, err_hbm,
                  srcbb, dstbb, wbb, rows, accv, zv, xbv, errb, bgv, acc_sh,
                  zsp, gsem, ssem):
    c = lax.axis_index("c")
    s = lax.axis_index("s")

    pltpu.sync_copy(bg_hbm, bgv)
    bsv = bgv[pl.ds(0, 16)]
    gsv = bgv[pl.ds(16, 16)]

    def _zero_row(r, _):
        for g in range(DH // 16):
            zv[r, pl.ds(g * 16, 16)] = jnp.zeros((16,), _f32)
        return 0
    lax.fori_loop(0, UCHUNK, _zero_row, 0)
    for k in range(NUCH):
        pltpu.sync_copy(zv, acc_sh.at[pl.ds(s * NODES_PER_TILE + k * UCHUNK,
                                            UCHUNK)])
    pltpu.sync_copy(z_hbm.at[pl.ds(c * NP + s * NODES_PER_TILE,
                                   NODES_PER_TILE)],
                    zsp.at[pl.ds(s * NODES_PER_TILE, NODES_PER_TILE)])
    plsc.subcore_barrier()

    def _fire_gather(g, b):
        pltpu.async_copy(zsp.at[srcbb.at[g]], rows.at[b], gsem.at[b])

    def _drain_gather(g, b):
        pltpu.make_async_copy(zsp.at[srcbb.at[g]], rows.at[b],
                              gsem.at[b]).wait()

    def _fire_scatter(g, b):
        pltpu.async_copy(rows.at[b], acc_sh.at[dstbb.at[g]], ssem.at[b],
                         add=True)

    def _drain_scatter(b):
        pltpu.make_async_copy(rows.at[b], acc_sh.at[dstbb.at[0]],
                              ssem.at[b]).wait()

    _dnums = lax.GatherDimensionNumbers(
        offset_dims=(), collapsed_slice_dims=(0,), start_index_map=(0,))

    def _mult(g, b):
        @plsc.parallel_loop(0, CHUNK // 16, 1, unroll=2)
        def _q(q):
            wv16 = wbb[g, pl.ds(q * 16, 16)]
            for t in range(16):
                wsp = lax.gather(
                    wv16, jnp.full((16, 1), t, jnp.int32), _dnums, (1,),
                    mode=lax.GatherScatterMode.PROMISE_IN_BOUNDS)
                e = q * 16 + t
                for f in range(DH // 16):
                    sl = pl.ds(f * 16, 16)
                    rows[b, e, sl] = rows[b, e, sl] * wsp

    for h in range(NSTAGE):
        base = s * CPT + h * QGRP
        pltpu.sync_copy(srcq_hbm.at[pl.ds(base, QGRP)], srcbb)
        pltpu.sync_copy(dstq_hbm.at[pl.ds(base, QGRP)], dstbb)
        pltpu.sync_copy(wq_hbm.at[pl.ds(base, QGRP)], wbb)
        for g0 in range(AHEAD):
            _fire_gather(g0, g0 % NBUF)

        @pl.loop(0, QGRP, step=NBUF)
        def _outer(gg):
            for b in range(NBUF):
                g = gg + b
                ga = g + AHEAD
                b2 = (b + AHEAD) % NBUF

                @pl.when(ga < QGRP)
                def _():
                    @pl.when(ga - NBUF >= 0)
                    def _():
                        _drain_scatter(b2)
                    _fire_gather(ga, b2)

                _drain_gather(g, b)
                _mult(g, b)
                _fire_scatter(g, b)

        for b in range(NBUF):
            _drain_scatter(b)
    plsc.subcore_barrier()

    def _upd(k, errv):
        row0 = s * NODES_PER_TILE + k * UCHUNK
        h1 = pltpu.async_copy(acc_sh.at[pl.ds(row0, UCHUNK)], accv,
                              gsem.at[0])
        h2 = pltpu.async_copy(z_hbm.at[pl.ds(c * NP + row0, UCHUNK)], zv,
                              gsem.at[1])
        h3 = pltpu.async_copy(xb_hbm.at[pl.ds(c * NP + row0, UCHUNK)], xbv,
                              gsem.at[2])
        h1.wait()
        h2.wait()
        h3.wait()

        def _row(r, ev):
            for g in range(DH // 16):
                sl = pl.ds(g * 16, 16)
                a = accv[r, sl]
                zz = zv[r, sl]
                xx = xbv[r, sl]
                zn = (1.0 - bsv) * zz + bsv * jnp.maximum(gsv * a + xx, 0.0)
                zv[r, sl] = zn
                dd = zn - zz
                ev = ev + dd * dd
            return ev
        errv = lax.fori_loop(0, UCHUNK, _row, errv)
        pltpu.sync_copy(zv, znew_hbm.at[pl.ds(c * NP + row0, UCHUNK)])
        return errv

    errv = lax.fori_loop(0, NUCH, _upd, jnp.zeros((16,), _f32))
    errb[...] = errv
    pltpu.sync_copy(errb, err_hbm.at[pl.ds((c * NS + s) * 16, 16)])


@jax.jit
def _sc_step(z, xb, srcq, dstq, wq, bg):
    mesh = plsc.VectorSubcoreMesh(core_axis_name="c", subcore_axis_name="s")
    return pl.kernel(
        _sc_step_body,
        out_type=(
            jax.ShapeDtypeStruct((2 * NP, DH), _f32),
            jax.ShapeDtypeStruct((NC * NS * 16,), _f32),
        ),
        mesh=mesh,
        compiler_params=pltpu.CompilerParams(use_tc_tiling_on_sc=False),
        scratch_types=[
            pltpu.VMEM((QGRP, CHUNK), jnp.int32),
            pltpu.VMEM((QGRP, CHUNK), jnp.int32),
            pltpu.VMEM((QGRP, CHUNK), _f32),
            pltpu.VMEM((NBUF, CHUNK, DH), _f32),
            pltpu.VMEM((UCHUNK, DH), _f32),
            pltpu.VMEM((UCHUNK, DH), _f32),
            pltpu.VMEM((UCHUNK, DH), _f32),
            pltpu.VMEM((16,), _f32),
            pltpu.VMEM((32,), _f32),
            pltpu.VMEM_SHARED((NP, DH), _f32),
            pltpu.VMEM_SHARED((NP, DH), _f32),
            pltpu.SemaphoreType.DMA((NBUF,)),
            pltpu.SemaphoreType.DMA((NBUF,)),
        ],
    )(z, xb, srcq, dstq, wq, bg)


def _enc_body(x_ref, wenc_ref, wbias_ref, b_ref, out_ref):
    h = jnp.dot(x_ref[...], wenc_ref[...], preferred_element_type=_f32)
    out_ref[...] = (
        jnp.dot(h, wbias_ref[...], preferred_element_type=_f32) + b_ref[...]
    )


@jax.jit
def _encoder(x, wenc_t, wbias_t, b):
    blk = 1000
    grid = N // blk
    return pl.pallas_call(
        _enc_body,
        grid=(grid,),
        in_specs=[
            pl.BlockSpec((blk, D), lambda i: (i, 0)),
            pl.BlockSpec((D, D), lambda i: (0, 0)),
            pl.BlockSpec((D, D), lambda i: (0, 0)),
            pl.BlockSpec((1, D), lambda i: (0, 0)),
        ],
        out_specs=pl.BlockSpec((blk, D), lambda i: (i, 0)),
        out_shape=jax.ShapeDtypeStruct((N, D), _f32),
    )(x, wenc_t, wbias_t, b)


def _dec_body(z0_ref, z1_ref, wdec_ref, b_ref, out_ref):
    h0 = jnp.maximum(z0_ref[...], 0.0)
    h1 = jnp.maximum(z1_ref[...], 0.0)
    out_ref[...] = (
        jnp.dot(h0, wdec_ref[:DH, :], preferred_element_type=_f32)
        + jnp.dot(h1, wdec_ref[DH:, :], preferred_element_type=_f32)
        + b_ref[...]
    )


@jax.jit
def _decoder(z0h, z1h, wdec_t, b):
    blk = 1000
    grid = N // blk
    return pl.pallas_call(
        _dec_body,
        grid=(grid,),
        in_specs=[
            pl.BlockSpec((blk, DH), lambda i: (i, 0)),
            pl.BlockSpec((blk, DH), lambda i: (i, 0)),
            pl.BlockSpec((D, D), lambda i: (0, 0)),
            pl.BlockSpec((1, D), lambda i: (0, 0)),
        ],
        out_specs=pl.BlockSpec((blk, D), lambda i: (i, 0)),
        out_shape=jax.ShapeDtypeStruct((N, D), _f32),
    )(z0h, z1h, wdec_t, b)


def kernel(x, edge_index, edge_weight, W_enc, W_bias, b_bias, W_dec, b_dec,
           beta, gamma):
    beta_s = jax.nn.sigmoid(beta)
    gamma_s = jax.nn.sigmoid(gamma)

    xb = _encoder(x, W_enc.T, W_bias.T, b_bias.reshape(1, D))
    xbflat = jnp.zeros((2 * NP, DH), _f32)
    xbflat = xbflat.at[0:N].set(xb[:, :DH]).at[NP:NP + N].set(xb[:, DH:])

    src = edge_index[0]
    dst = edge_index[1]
    pad = E_PAD - E
    srcp = jnp.concatenate([src, jnp.zeros((pad,), jnp.int32)])
    dstp = jnp.concatenate([dst, jnp.zeros((pad,), jnp.int32)])
    wp = jnp.concatenate([edge_weight, jnp.zeros((pad,), _f32)])
    srcq = srcp.reshape(NS * CPT, CHUNK)
    dstq = dstp.reshape(NS * CPT, CHUNK)
    wq = wp.reshape(NS * CPT, CHUNK)
    bg = jnp.concatenate([jnp.broadcast_to(beta_s, (16,)),
                          jnp.broadcast_to(gamma_s, (16,))]).astype(_f32)

    def step(z):
        znew, errparts = _sc_step(z, xbflat, srcq, dstq, wq, bg)
        return znew, jnp.sum(errparts)

    z0 = jnp.zeros((2 * NP, DH), _f32)
    z1, e1 = step(z0)

    def cond(state):
        _, errsq, it = state
        return jnp.logical_and(errsq > jnp.float32(TOL) * jnp.float32(TOL),
                               it < MAX_ITER)

    def body(state):
        z, _, it = state
        znew, errsq = step(z)
        return (znew, errsq, it + 1)

    z_star, _, _ = lax.while_loop(cond, body, (z1, e1, jnp.int32(1)))

    z = z_star
    for _ in range(PHANTOM_GRAD):
        z, _ = step(z)

    return _decoder(z[0:N], z[NP:NP + N], W_dec.T, b_dec.reshape(1, D))
